# Initial kernel scaffold; baseline (speedup 1.0000x reference)
import jax
import jax.numpy as jnp
from jax.experimental import pallas as pl


def kernel(x, W_router, expert_bias, shared_Wg, shared_Wu, shared_Wd, Wg, Wu, Wd):
    raise NotImplementedError("write your pallas kernel here")


# speedup vs baseline: 1.2947x; 1.2947x over previous
import functools

import jax
import jax.numpy as jnp
from jax import lax
from jax.experimental import pallas as pl

D_MODEL = 768
D_FFN = 2048
N_EXPERTS = 8
SEQ = 2048
FB = 512
NEG = -1e30


def _router_body(x_ref, wrt_ref, bias_ref, w_ref):
    x = x_ref[...]
    logits = jnp.dot(x, wrt_ref[...], preferred_element_type=jnp.float32)
    lane = lax.broadcasted_iota(jnp.int32, logits.shape, 1)
    valid = lane < N_EXPERTS
    l = jnp.where(valid, logits + bias_ref[...], NEG)
    m1 = jnp.max(l, axis=1, keepdims=True)
    a1 = jnp.min(jnp.where(l == m1, lane, 128), axis=1, keepdims=True)
    oh1 = lane == a1
    l2 = jnp.where(oh1, NEG, l)
    m2 = jnp.max(l2, axis=1, keepdims=True)
    a2 = jnp.min(jnp.where(l2 == m2, lane, 128), axis=1, keepdims=True)
    oh2 = lane == a2
    e21 = jnp.exp(m2 - m1)
    w1 = 1.0 / (1.0 + e21)
    w2 = 1.0 - w1
    w_ref[...] = jnp.where(oh1, w1, 0.0) + jnp.where(oh2, w2, 0.0)


def _shared_body(x_ref, wg_ref, wu_ref, wd_ref, out_ref):
    f = pl.program_id(0)
    x = x_ref[...]
    g = lax.dot_general(x, wg_ref[0], (((1,), (1,)), ((), ())),
                        preferred_element_type=jnp.float32)
    u = lax.dot_general(x, wu_ref[0], (((1,), (1,)), ((), ())),
                        preferred_element_type=jnp.float32)
    h = g * jax.nn.sigmoid(g) * u
    contrib = lax.dot_general(h, wd_ref[0], (((1,), (1,)), ((), ())),
                              preferred_element_type=jnp.float32)

    @pl.when(f == 0)
    def _():
        out_ref[...] = jnp.zeros_like(out_ref)

    out_ref[...] += contrib


def _experts_body(x_ref, w_ref, shared_ref, wg_ref, wu_ref, wd_ref, out_ref):
    e = pl.program_id(0)
    f = pl.program_id(1)
    x = x_ref[...]
    g = lax.dot_general(x, wg_ref[0], (((1,), (1,)), ((), ())),
                        preferred_element_type=jnp.float32)
    u = lax.dot_general(x, wu_ref[0], (((1,), (1,)), ((), ())),
                        preferred_element_type=jnp.float32)
    h = g * jax.nn.sigmoid(g) * u
    lane = lax.broadcasted_iota(jnp.int32, w_ref.shape, 1)
    wcol = jnp.sum(jnp.where(lane == e, w_ref[...], 0.0), axis=1,
                   keepdims=True)
    contrib = lax.dot_general(h * wcol, wd_ref[0], (((1,), (1,)), ((), ())),
                              preferred_element_type=jnp.float32)

    @pl.when(jnp.logical_and(e == 0, f == 0))
    def _():
        out_ref[...] = shared_ref[...]

    out_ref[...] += contrib


def kernel(x, W_router, expert_bias, shared_Wg, shared_Wu, shared_Wd, Wg, Wu, Wd):
    b, s, d = x.shape
    xf = x.reshape(-1, d)
    T = xf.shape[0]

    wrt = jnp.zeros((d, 128), x.dtype).at[:, :N_EXPERTS].set(W_router.T)
    bias = jnp.zeros((1, 128), x.dtype).at[0, :N_EXPERTS].set(expert_bias)

    w = pl.pallas_call(
        _router_body,
        out_shape=jax.ShapeDtypeStruct((T, 128), jnp.float32),
    )(xf, wrt, bias)

    nf = D_FFN // FB
    shared_out = pl.pallas_call(
        _shared_body,
        grid=(nf,),
        in_specs=[
            pl.BlockSpec((T, d), lambda f: (0, 0)),
            pl.BlockSpec((1, FB, d), lambda f: (0, f, 0)),
            pl.BlockSpec((1, FB, d), lambda f: (0, f, 0)),
            pl.BlockSpec((1, d, FB), lambda f: (0, 0, f)),
        ],
        out_specs=pl.BlockSpec((T, d), lambda f: (0, 0)),
        out_shape=jax.ShapeDtypeStruct((T, d), jnp.float32),
    )(xf, shared_Wg, shared_Wu, shared_Wd)

    out = pl.pallas_call(
        _experts_body,
        grid=(N_EXPERTS, nf),
        in_specs=[
            pl.BlockSpec((T, d), lambda e, f: (0, 0)),
            pl.BlockSpec((T, 128), lambda e, f: (0, 0)),
            pl.BlockSpec((T, d), lambda e, f: (0, 0)),
            pl.BlockSpec((1, FB, d), lambda e, f: (e, f, 0)),
            pl.BlockSpec((1, FB, d), lambda e, f: (e, f, 0)),
            pl.BlockSpec((1, d, FB), lambda e, f: (e, 0, f)),
        ],
        out_specs=pl.BlockSpec((T, d), lambda e, f: (0, 0)),
        out_shape=jax.ShapeDtypeStruct((T, d), jnp.float32),
    )(xf, w, shared_out, Wg, Wu, Wd)

    return out.reshape(b, s, d)
